# baseline (speedup 1.0000x reference)
import functools

import jax
import jax.numpy as jnp
from jax import lax
from jax.experimental import pallas as pl
from jax.experimental.pallas import tpu as pltpu
from jax.experimental.pallas import tpu_sc as plsc

N_ROWS = 16384
DIM = 2048
NC = 2
NS = 16
NW = NC * NS
L = 16

ROWS_PER_W = N_ROWS // NW
BLK = 8
N_BLKS = ROWS_PER_W // BLK
GRPS = DIM // L

_mesh = plsc.VectorSubcoreMesh(
    core_axis_name="c", subcore_axis_name="s", num_cores=NC, num_subcores=NS
)


@functools.partial(
    pl.kernel,
    out_type=jax.ShapeDtypeStruct((N_ROWS, DIM), jnp.float32),
    mesh=_mesh,
    scratch_types=[
        pltpu.VMEM((DIM,), jnp.int32),
        pltpu.VMEM((BLK, DIM), jnp.float32),
        pltpu.VMEM((BLK, DIM), jnp.float32),
    ],
    compiler_params=pltpu.CompilerParams(needs_layout_passes=False),
)
def _reorder_sc(x_hbm, perm_hbm, y_hbm, perm_v, in_v, out_v):
    wid = lax.axis_index("s") * NC + lax.axis_index("c")
    base0 = wid * ROWS_PER_W

    pltpu.sync_copy(perm_hbm, perm_v)

    def do_block(b, _):
        base = base0 + b * BLK
        pltpu.sync_copy(x_hbm.at[pl.ds(base, BLK)], in_v)

        def do_row(r, _):
            rvec = jnp.full((L,), r, jnp.int32)

            def do_grp(j, _):
                off = pl.multiple_of(j * L, L)
                idx = perm_v[pl.ds(off, L)]
                out_v[r, pl.ds(off, L)] = plsc.load_gather(in_v, [rvec, idx])
                return ()

            lax.fori_loop(0, GRPS, do_grp, (), unroll=4)
            return ()

        lax.fori_loop(0, BLK, do_row, ())
        pltpu.sync_copy(out_v, y_hbm.at[pl.ds(base, BLK)])
        return ()

    lax.fori_loop(0, N_BLKS, do_block, ())


def kernel(x, logp, randperm):
    y = _reorder_sc(x, randperm)
    if logp is None:
        return y
    return (y, logp)


# speedup vs baseline: 2.5797x; 2.5797x over previous
import functools

import jax
import jax.numpy as jnp
from jax import lax
from jax.experimental import pallas as pl
from jax.experimental.pallas import tpu as pltpu
from jax.experimental.pallas import tpu_sc as plsc

N_ROWS = 16384
DIM = 2048
NC = 2
NS = 16
NW = NC * NS
L = 16

ROWS_PER_W = N_ROWS // NW
BLK = 8
N_BLKS = ROWS_PER_W // BLK
GRPS = DIM // L

_mesh = plsc.VectorSubcoreMesh(
    core_axis_name="c", subcore_axis_name="s", num_cores=NC, num_subcores=NS
)


@functools.partial(
    pl.kernel,
    out_type=jax.ShapeDtypeStruct((N_ROWS, DIM), jnp.float32),
    mesh=_mesh,
    scratch_types=[
        pltpu.VMEM((DIM,), jnp.int32),
        pltpu.VMEM((BLK, DIM), jnp.float32),
        pltpu.VMEM((BLK, DIM), jnp.float32),
        pltpu.VMEM((BLK, DIM), jnp.float32),
        pltpu.VMEM((BLK, DIM), jnp.float32),
        pltpu.SemaphoreType.DMA,
        pltpu.SemaphoreType.DMA,
        pltpu.SemaphoreType.DMA,
        pltpu.SemaphoreType.DMA,
    ],
    compiler_params=pltpu.CompilerParams(needs_layout_passes=False),
)
def _reorder_sc(x_hbm, perm_hbm, y_hbm, perm_v, in0, in1, out0, out1,
                sin0, sin1, sout0, sout1):
    wid = lax.axis_index("s") * NC + lax.axis_index("c")
    base0 = wid * ROWS_PER_W

    in_bufs = (in0, in1)
    out_bufs = (out0, out1)
    sin = (sin0, sin1)
    sout = (sout0, sout1)

    pltpu.sync_copy(perm_hbm, perm_v)

    rvecs = [jnp.full((L,), r, jnp.int32) for r in range(BLK)]

    def in_copy(b, q):
        return pltpu.make_async_copy(
            x_hbm.at[pl.ds(base0 + b * BLK, BLK)], in_bufs[q], sin[q]
        )

    def out_copy(b, q):
        return pltpu.make_async_copy(
            out_bufs[q], y_hbm.at[pl.ds(base0 + b * BLK, BLK)], sout[q]
        )

    def compute(q):
        in_v = in_bufs[q]
        out_v = out_bufs[q]

        def do_grp(j, _):
            off = pl.multiple_of(j * L, L)
            idx = perm_v[pl.ds(off, L)]
            for r in range(BLK):
                out_v[r, pl.ds(off, L)] = plsc.load_gather(in_v, [rvecs[r], idx])
            return ()

        lax.fori_loop(0, GRPS, do_grp, (), unroll=2)

    in_copy(0, 0).start()
    in_copy(1, 1).start()

    def pair_body(p, _):
        for q in (0, 1):
            b = 2 * p + q
            in_copy(b, q).wait()

            @pl.when(b >= 2)
            def _wait_out():
                out_copy(b - 2, q).wait()

            compute(q)
            out_copy(b, q).start()

            @pl.when(b + 2 < N_BLKS)
            def _next_in():
                in_copy(b + 2, q).start()

        return ()

    lax.fori_loop(0, N_BLKS // 2, pair_body, ())
    out_copy(N_BLKS - 2, 0).wait()
    out_copy(N_BLKS - 1, 1).wait()


def kernel(x, logp, randperm):
    y = _reorder_sc(x, randperm)
    if logp is None:
        return y
    return (y, logp)


# speedup vs baseline: 6.2542x; 2.4244x over previous
import functools

import jax
import jax.numpy as jnp
from jax import lax
from jax.experimental import pallas as pl
from jax.experimental.pallas import tpu as pltpu
from jax.experimental.pallas import tpu_sc as plsc

N_ROWS = 16384
DIM = 2048
NC = 2
NS = 16
NW = NC * NS
L = 16

ROWS_PER_W = N_ROWS // NW
BLK = 8
N_BLKS = ROWS_PER_W // BLK
GRPS = DIM // L

_mesh = plsc.VectorSubcoreMesh(
    core_axis_name="c", subcore_axis_name="s", num_cores=NC, num_subcores=NS
)


@functools.partial(
    pl.kernel,
    out_type=jax.ShapeDtypeStruct((N_ROWS, DIM), jnp.float32),
    mesh=_mesh,
    scratch_types=[
        pltpu.VMEM((DIM + L,), jnp.int32),
        pltpu.VMEM((BLK, DIM), jnp.float32),
        pltpu.VMEM((BLK, DIM), jnp.float32),
        pltpu.VMEM((BLK, DIM), jnp.float32),
        pltpu.VMEM((BLK, DIM), jnp.float32),
        pltpu.SemaphoreType.DMA,
        pltpu.SemaphoreType.DMA,
        pltpu.SemaphoreType.DMA,
        pltpu.SemaphoreType.DMA,
    ],
    compiler_params=pltpu.CompilerParams(needs_layout_passes=False),
)
def _reorder_sc(x_hbm, perm_hbm, y_hbm, perm_v, in0, in1, out0, out1,
                sin0, sin1, sout0, sout1):
    wid = lax.axis_index("s") * NC + lax.axis_index("c")
    base0 = wid * ROWS_PER_W

    in_bufs = (in0, in1)
    out_bufs = (out0, out1)
    sin = (sin0, sin1)
    sout = (sout0, sout1)

    pltpu.sync_copy(perm_hbm, perm_v.at[pl.ds(0, DIM)])

    rvecs = [jnp.full((L,), r, jnp.int32) for r in range(BLK)]

    def in_copy(b, q):
        return pltpu.make_async_copy(
            x_hbm.at[pl.ds(base0 + b * BLK, BLK)], in_bufs[q], sin[q]
        )

    def out_copy(b, q):
        return pltpu.make_async_copy(
            out_bufs[q], y_hbm.at[pl.ds(base0 + b * BLK, BLK)], sout[q]
        )

    def compute(q):
        in_v = in_bufs[q]
        out_v = out_bufs[q]

        def do_grp(j, idx_cur):
            off_next = pl.multiple_of(j * L + L, L)
            idx_next = perm_v[pl.ds(off_next, L)]
            vals = [plsc.load_gather(in_v, [rvecs[r], idx_cur])
                    for r in range(BLK)]
            off = pl.multiple_of(j * L, L)
            for r in range(BLK):
                out_v[r, pl.ds(off, L)] = vals[r]
            return idx_next

        idx0 = perm_v[pl.ds(0, L)]
        lax.fori_loop(0, GRPS, do_grp, idx0, unroll=2)

    in_copy(0, 0).start()
    in_copy(1, 1).start()

    def pair_body(p, _):
        for q in (0, 1):
            b = 2 * p + q
            in_copy(b, q).wait()

            @pl.when(b >= 2)
            def _wait_out():
                out_copy(b - 2, q).wait()

            compute(q)
            out_copy(b, q).start()

            @pl.when(b + 2 < N_BLKS)
            def _next_in():
                in_copy(b + 2, q).start()

        return ()

    lax.fori_loop(0, N_BLKS // 2, pair_body, ())
    out_copy(N_BLKS - 2, 0).wait()
    out_copy(N_BLKS - 1, 1).wait()


def kernel(x, logp, randperm):
    y = _reorder_sc(x, randperm)
    if logp is None:
        return y
    return (y, logp)
